# baseline (speedup 1.0000x reference)
import functools

import jax
import jax.numpy as jnp
from jax import lax
from jax.experimental import pallas as pl
from jax.experimental.pallas import tpu as pltpu
from jax.experimental.pallas import tpu_sc as plsc

_N = 10000
_E = 320000
_D = 128
_DH = _D // 2
_DA = _DH + 8

_NC = 2
_NS = 16

_CHUNK = 128
_CHUNKS_PER_T = 160
_WIN = 10
_ECHUNKS = _E // _CHUNK
_N_PAD = 10112
_ROWS_PER_TILE = _N_PAD // _NS
_ACC_ROWS = _N
_LAST_ROWS = _ACC_ROWS - 15 * _ROWS_PER_TILE
_FULL_CHUNKS = _CHUNKS_PER_T
_LAST_CHUNKS = _ECHUNKS - 15 * _CHUNKS_PER_T


def _sc_aggregate_body(y0_hbm, y1_hbm, ei_hbm, acc_out,
                       srcw_v, dstw_v, rows_v0, rows_v1,
                       stage_acc, acc_sh, gsem0, gsem1, wsem):
    c = lax.axis_index("c")
    s = lax.axis_index("s")
    base = s * _ROWS_PER_TILE
    cbase = s * _CHUNKS_PER_T

    z16 = jnp.zeros((16,), jnp.float32)

    def _zrow(i, carry):
        for k in range(_DA // 16):
            stage_acc[i, pl.ds(k * 16, 16)] = z16
        if _DA % 16:
            stage_acc[i, pl.ds(_DA - 16, 16)] = z16
        return carry

    lax.fori_loop(0, _ROWS_PER_TILE, _zrow, 0)

    @pl.when(s < 15)
    def _():
        pltpu.sync_copy(stage_acc, acc_sh.at[pl.ds(base, _ROWS_PER_TILE)])

    @pl.when(s == 15)
    def _():
        pltpu.sync_copy(stage_acc.at[pl.ds(0, _LAST_ROWS)],
                        acc_sh.at[pl.ds(15 * _ROWS_PER_TILE, _LAST_ROWS)])

    plsc.subcore_barrier()

    def _run(y_hbm):
        rows = (rows_v0, rows_v1)
        sems = (gsem0, gsem1)

        def _stage(w, q):
            pltpu.async_copy(ei_hbm.at[0, pl.ds(cbase + w * _WIN, _WIN)],
                             srcw_v.at[pl.ds(q * _WIN, _WIN)], wsem)
            pltpu.async_copy(ei_hbm.at[1, pl.ds(cbase + w * _WIN, _WIN)],
                             dstw_v.at[pl.ds(q * _WIN, _WIN)], wsem)

        def _drain_stage():
            pltpu.make_async_copy(ei_hbm.at[0, pl.ds(0, _WIN)],
                                  srcw_v.at[pl.ds(0, _WIN)], wsem).wait()
            pltpu.make_async_copy(ei_hbm.at[1, pl.ds(0, _WIN)],
                                  dstw_v.at[pl.ds(0, _WIN)], wsem).wait()

        def _start(r, b):
            pltpu.async_copy(y_hbm.at[srcw_v.at[r]], rows[b], sems[b])

        def _scatter(r, b):
            pltpu.make_async_copy(y_hbm.at[srcw_v.at[0]], rows[b],
                                  sems[b]).wait()
            pltpu.sync_copy(rows[b], acc_sh.at[dstw_v.at[r]], add=True)

        nwin = jnp.where(s == 15, _LAST_CHUNKS // _WIN, _FULL_CHUNKS // _WIN)
        _stage(0, 0)
        _drain_stage()

        def win(w, carry):
            q = lax.rem(w, 2)
            rbase = q * _WIN

            @pl.when(w + 1 < nwin)
            def _():
                _stage(w + 1, 1 - q)

            _start(rbase, 0)

            def pair(p, carry2):
                r0 = rbase + 2 * p
                _start(r0 + 1, 1)
                _scatter(r0, 0)

                @pl.when(p + 1 < _WIN // 2)
                def _():
                    _start(r0 + 2, 0)

                _scatter(r0 + 1, 1)
                return carry2

            lax.fori_loop(0, _WIN // 2, pair, 0)

            @pl.when(w + 1 < nwin)
            def _():
                _drain_stage()

            return carry

        lax.fori_loop(0, nwin, win, 0)

    @pl.when(c == 0)
    def _():
        _run(y0_hbm)

    @pl.when(c == 1)
    def _():
        _run(y1_hbm)

    plsc.subcore_barrier()

    @pl.when(s < 15)
    def _():
        pltpu.sync_copy(acc_sh.at[pl.ds(base, _ROWS_PER_TILE)], stage_acc)
        pltpu.sync_copy(stage_acc, acc_out.at[c, pl.ds(base, _ROWS_PER_TILE)])

    @pl.when(s == 15)
    def _():
        pltpu.sync_copy(acc_sh.at[pl.ds(15 * _ROWS_PER_TILE, _LAST_ROWS)],
                        stage_acc.at[pl.ds(0, _LAST_ROWS)])
        pltpu.sync_copy(stage_acc.at[pl.ds(0, _LAST_ROWS)],
                        acc_out.at[c, pl.ds(15 * _ROWS_PER_TILE, _LAST_ROWS)])


_sc_aggregate = functools.partial(
    pl.kernel,
    out_type=jax.ShapeDtypeStruct((_NC, _ACC_ROWS, _DA), jnp.float32),
    mesh=plsc.VectorSubcoreMesh(core_axis_name="c", subcore_axis_name="s"),
    compiler_params=pltpu.CompilerParams(use_tc_tiling_on_sc=False),
    scratch_types=[
        pltpu.VMEM((2 * _WIN, _CHUNK), jnp.int32),
        pltpu.VMEM((2 * _WIN, _CHUNK), jnp.int32),
        pltpu.VMEM((_CHUNK, _DA), jnp.float32),
        pltpu.VMEM((_CHUNK, _DA), jnp.float32),
        pltpu.VMEM((_ROWS_PER_TILE, _DA), jnp.float32),
        pltpu.VMEM_SHARED((_ACC_ROWS, _DA), jnp.float32),
        pltpu.SemaphoreType.DMA,
        pltpu.SemaphoreType.DMA,
        pltpu.SemaphoreType.DMA,
    ],
)(_sc_aggregate_body)


def _neigh_mm_body(x_ref, w_ref, y0_ref, y1_ref):
    y = jnp.dot(x_ref[...], w_ref[...], preferred_element_type=jnp.float32)
    ones = jnp.ones((_N, _DA - _DH), jnp.float32)
    y0_ref[...] = jnp.concatenate([y[:, :_DH], ones], axis=1)
    y1_ref[...] = jnp.concatenate([y[:, _DH:], ones], axis=1)


def _neigh_mm(x, w_neigh_t):
    return pl.pallas_call(
        _neigh_mm_body,
        out_shape=[jax.ShapeDtypeStruct((_N, _DA), jnp.float32),
                   jax.ShapeDtypeStruct((_N, _DA), jnp.float32)],
    )(x, w_neigh_t)


def _combine_body(x_ref, w_ref, b_ref, acc_ref, out_ref):
    a0 = acc_ref[0]
    a1 = acc_ref[1]
    deg = a0[:, _DH:_DH + 1]
    r = 1.0 / jnp.maximum(deg, 1.0)
    h = jnp.concatenate([a0[:, :_DH], a1[:, :_DH]], axis=1) * r
    out_ref[...] = (
        jnp.dot(x_ref[...], w_ref[...], preferred_element_type=jnp.float32)
        + h + b_ref[...]
    )


def _combine(x, w_self_t, b2d, acc):
    return pl.pallas_call(
        _combine_body,
        out_shape=jax.ShapeDtypeStruct((_N, _D), jnp.float32),
    )(x, w_self_t, b2d, acc)


def kernel(x, edge_index, W_self, W_neigh, b):
    ei = edge_index.reshape(2, _ECHUNKS, _CHUNK)
    y0, y1 = _neigh_mm(x, W_neigh.T)
    acc = _sc_aggregate(y0, y1, ei)
    return _combine(x, W_self.T, b.reshape(1, _D), acc)


# speedup vs baseline: 1.0275x; 1.0275x over previous
import functools

import jax
import jax.numpy as jnp
from jax import lax
from jax.experimental import pallas as pl
from jax.experimental.pallas import tpu as pltpu
from jax.experimental.pallas import tpu_sc as plsc

_N = 10000
_E = 320000
_D = 128
_DH = _D // 2
_DA = _DH + 8

_NC = 2
_NS = 16

_CHUNK = 128
_CHUNKS_PER_T = 160
_WIN = 20
_ECHUNKS = _E // _CHUNK
_N_PAD = 10112
_ROWS_PER_TILE = _N_PAD // _NS
_ACC_ROWS = _N
_LAST_ROWS = _ACC_ROWS - 15 * _ROWS_PER_TILE
_FULL_CHUNKS = _CHUNKS_PER_T
_LAST_CHUNKS = _ECHUNKS - 15 * _CHUNKS_PER_T


def _sc_aggregate_body(y0_hbm, y1_hbm, ei_hbm, acc_out,
                       srcw_v, dstw_v, rows_v0, rows_v1,
                       stage_acc, acc_sh, gsem0, gsem1, wsem):
    c = lax.axis_index("c")
    s = lax.axis_index("s")
    base = s * _ROWS_PER_TILE
    cbase = s * _CHUNKS_PER_T

    z16 = jnp.zeros((16,), jnp.float32)

    def _zrow(i, carry):
        for k in range(_DA // 16):
            stage_acc[i, pl.ds(k * 16, 16)] = z16
        if _DA % 16:
            stage_acc[i, pl.ds(_DA - 16, 16)] = z16
        return carry

    lax.fori_loop(0, _ROWS_PER_TILE, _zrow, 0)

    @pl.when(s < 15)
    def _():
        pltpu.sync_copy(stage_acc, acc_sh.at[pl.ds(base, _ROWS_PER_TILE)])

    @pl.when(s == 15)
    def _():
        pltpu.sync_copy(stage_acc.at[pl.ds(0, _LAST_ROWS)],
                        acc_sh.at[pl.ds(15 * _ROWS_PER_TILE, _LAST_ROWS)])

    plsc.subcore_barrier()

    def _run(y_hbm):
        rows = (rows_v0, rows_v1)
        sems = (gsem0, gsem1)

        def _stage(w, q):
            pltpu.async_copy(ei_hbm.at[0, pl.ds(cbase + w * _WIN, _WIN)],
                             srcw_v.at[pl.ds(q * _WIN, _WIN)], wsem)
            pltpu.async_copy(ei_hbm.at[1, pl.ds(cbase + w * _WIN, _WIN)],
                             dstw_v.at[pl.ds(q * _WIN, _WIN)], wsem)

        def _drain_stage():
            pltpu.make_async_copy(ei_hbm.at[0, pl.ds(0, _WIN)],
                                  srcw_v.at[pl.ds(0, _WIN)], wsem).wait()
            pltpu.make_async_copy(ei_hbm.at[1, pl.ds(0, _WIN)],
                                  dstw_v.at[pl.ds(0, _WIN)], wsem).wait()

        def _start(r, b):
            pltpu.async_copy(y_hbm.at[srcw_v.at[r]], rows[b], sems[b])

        def _scatter(r, b):
            pltpu.make_async_copy(y_hbm.at[srcw_v.at[0]], rows[b],
                                  sems[b]).wait()
            pltpu.sync_copy(rows[b], acc_sh.at[dstw_v.at[r]], add=True)

        nwin = jnp.where(s == 15, _LAST_CHUNKS // _WIN, _FULL_CHUNKS // _WIN)
        _stage(0, 0)
        _drain_stage()

        def win(w, carry):
            q = lax.rem(w, 2)
            rbase = q * _WIN

            @pl.when(w + 1 < nwin)
            def _():
                _stage(w + 1, 1 - q)

            _start(rbase, 0)

            def pair(p, carry2):
                r0 = rbase + 2 * p
                _start(r0 + 1, 1)
                _scatter(r0, 0)

                @pl.when(p + 1 < _WIN // 2)
                def _():
                    _start(r0 + 2, 0)

                _scatter(r0 + 1, 1)
                return carry2

            lax.fori_loop(0, _WIN // 2, pair, 0)

            @pl.when(w + 1 < nwin)
            def _():
                _drain_stage()

            return carry

        lax.fori_loop(0, nwin, win, 0)

    @pl.when(c == 0)
    def _():
        _run(y0_hbm)

    @pl.when(c == 1)
    def _():
        _run(y1_hbm)

    plsc.subcore_barrier()

    @pl.when(s < 15)
    def _():
        pltpu.sync_copy(acc_sh.at[pl.ds(base, _ROWS_PER_TILE)], stage_acc)
        pltpu.sync_copy(stage_acc, acc_out.at[c, pl.ds(base, _ROWS_PER_TILE)])

    @pl.when(s == 15)
    def _():
        pltpu.sync_copy(acc_sh.at[pl.ds(15 * _ROWS_PER_TILE, _LAST_ROWS)],
                        stage_acc.at[pl.ds(0, _LAST_ROWS)])
        pltpu.sync_copy(stage_acc.at[pl.ds(0, _LAST_ROWS)],
                        acc_out.at[c, pl.ds(15 * _ROWS_PER_TILE, _LAST_ROWS)])


_sc_aggregate = functools.partial(
    pl.kernel,
    out_type=jax.ShapeDtypeStruct((_NC, _ACC_ROWS, _DA), jnp.float32),
    mesh=plsc.VectorSubcoreMesh(core_axis_name="c", subcore_axis_name="s"),
    compiler_params=pltpu.CompilerParams(use_tc_tiling_on_sc=False),
    scratch_types=[
        pltpu.VMEM((2 * _WIN, _CHUNK), jnp.int32),
        pltpu.VMEM((2 * _WIN, _CHUNK), jnp.int32),
        pltpu.VMEM((_CHUNK, _DA), jnp.float32),
        pltpu.VMEM((_CHUNK, _DA), jnp.float32),
        pltpu.VMEM((_ROWS_PER_TILE, _DA), jnp.float32),
        pltpu.VMEM_SHARED((_ACC_ROWS, _DA), jnp.float32),
        pltpu.SemaphoreType.DMA,
        pltpu.SemaphoreType.DMA,
        pltpu.SemaphoreType.DMA,
    ],
)(_sc_aggregate_body)


def _neigh_mm_body(x_ref, w_ref, y0_ref, y1_ref):
    y = jnp.dot(x_ref[...], w_ref[...], preferred_element_type=jnp.float32)
    ones = jnp.ones((_N, _DA - _DH), jnp.float32)
    y0_ref[...] = jnp.concatenate([y[:, :_DH], ones], axis=1)
    y1_ref[...] = jnp.concatenate([y[:, _DH:], ones], axis=1)


def _neigh_mm(x, w_neigh_t):
    return pl.pallas_call(
        _neigh_mm_body,
        out_shape=[jax.ShapeDtypeStruct((_N, _DA), jnp.float32),
                   jax.ShapeDtypeStruct((_N, _DA), jnp.float32)],
    )(x, w_neigh_t)


def _combine_body(x_ref, w_ref, b_ref, acc_ref, out_ref):
    a0 = acc_ref[0]
    a1 = acc_ref[1]
    deg = a0[:, _DH:_DH + 1]
    r = 1.0 / jnp.maximum(deg, 1.0)
    h = jnp.concatenate([a0[:, :_DH], a1[:, :_DH]], axis=1) * r
    out_ref[...] = (
        jnp.dot(x_ref[...], w_ref[...], preferred_element_type=jnp.float32)
        + h + b_ref[...]
    )


def _combine(x, w_self_t, b2d, acc):
    return pl.pallas_call(
        _combine_body,
        out_shape=jax.ShapeDtypeStruct((_N, _D), jnp.float32),
    )(x, w_self_t, b2d, acc)


def kernel(x, edge_index, W_self, W_neigh, b):
    ei = edge_index.reshape(2, _ECHUNKS, _CHUNK)
    y0, y1 = _neigh_mm(x, W_neigh.T)
    acc = _sc_aggregate(y0, y1, ei)
    return _combine(x, W_self.T, b.reshape(1, _D), acc)
